# Initial kernel scaffold; baseline (speedup 1.0000x reference)
import jax
import jax.numpy as jnp
from jax.experimental import pallas as pl


def kernel(x, edge_index, W1, b1, W2, b2):
    raise NotImplementedError("write your pallas kernel here")


# speedup vs baseline: 11.3503x; 11.3503x over previous
import functools

import jax
import jax.numpy as jnp
from jax import lax
from jax.experimental import pallas as pl
from jax.experimental.pallas import tpu as pltpu
from jax.experimental.pallas import tpu_sc as plsc

N = 10000
E = 320000
D = 128

NC = 2
NS = 16
NW = NC * NS
CH = 128
EPW = 10112
NCH = EPW // CH
E_PAD = NW * EPW
RPT = 640
ACC = NS * RPT

_mesh = plsc.VectorSubcoreMesh(core_axis_name="c", subcore_axis_name="s")


@functools.partial(
    pl.kernel, mesh=_mesh,
    out_type=jax.ShapeDtypeStruct((NW, ACC), jnp.float32),
    scratch_types=[
        pltpu.VMEM((CH,), jnp.int32),
        pltpu.VMEM((ACC,), jnp.float32),
    ],
    compiler_params=pltpu.CompilerParams(needs_layout_passes=False),
)
def _deg_kernel(dst_hbm, out_hbm, idxb, hist):
    c = lax.axis_index("c")
    s = lax.axis_index("s")
    w = c * NS + s
    ones = jnp.ones((16,), jnp.float32)

    def _zero(i, carry):
        hist[pl.ds(i * 16, 16)] = jnp.zeros((16,), jnp.float32)
        return carry

    lax.fori_loop(0, ACC // 16, _zero, 0)

    e0 = w * EPW

    def _body(i, carry):
        pltpu.sync_copy(dst_hbm.at[pl.ds(e0 + i * CH, CH)], idxb)
        for k in range(CH // 16):
            idx = idxb[pl.ds(k * 16, 16)]
            plsc.addupdate_scatter(hist, [idx], ones)
        return carry

    lax.fori_loop(0, NCH, _body, 0)
    pltpu.sync_copy(hist, out_hbm.at[w])


@functools.partial(
    pl.kernel, mesh=_mesh,
    out_type=jax.ShapeDtypeStruct((NC, ACC, D), jnp.float32),
    scratch_types=[
        pltpu.VMEM((CH,), jnp.int32),
        pltpu.VMEM((CH,), jnp.int32),
        pltpu.VMEM((CH, D), jnp.float32),
        pltpu.VMEM_SHARED((ACC, D), jnp.float32),
    ],
)
def _scat_kernel(hn_hbm, src_hbm, dst_hbm, out_hbm, srcb, dstb, rows, acc):
    c = lax.axis_index("c")
    s = lax.axis_index("s")

    def _zero(r, carry):
        for j in range(D // 16):
            rows[r, pl.ds(j * 16, 16)] = jnp.zeros((16,), jnp.float32)
        return carry

    lax.fori_loop(0, CH, _zero, 0)
    for z in range(RPT // CH):
        pltpu.sync_copy(rows, acc.at[pl.ds(s * RPT + z * CH, CH)])
    plsc.subcore_barrier()

    e0 = (c * NS + s) * EPW

    def _body(i, carry):
        off = e0 + i * CH
        pltpu.sync_copy(src_hbm.at[pl.ds(off, CH)], srcb)
        pltpu.sync_copy(dst_hbm.at[pl.ds(off, CH)], dstb)
        pltpu.sync_copy(hn_hbm.at[srcb], rows)
        pltpu.sync_copy(rows, acc.at[dstb], add=True)
        return carry

    lax.fori_loop(0, NCH, _body, 0)
    plsc.subcore_barrier()
    pltpu.sync_copy(acc.at[pl.ds(s * RPT, RPT)],
                    out_hbm.at[c, pl.ds(s * RPT, RPT)])


MB = 2000
GRID = N // MB


def _lin1_body(x_ref, w_ref, hist_ref, o_ref):
    dinv = lax.rsqrt(jnp.sum(hist_ref[...], axis=1, keepdims=True) + 1.0)
    h = jnp.dot(x_ref[...], w_ref[...], preferred_element_type=jnp.float32)
    o_ref[...] = h * dinv


_lin1 = pl.pallas_call(
    _lin1_body,
    grid=(GRID,),
    in_specs=[
        pl.BlockSpec((MB, D), lambda i: (i, 0)),
        pl.BlockSpec((D, D), lambda i: (0, 0)),
        pl.BlockSpec((MB, NW), lambda i: (i, 0)),
    ],
    out_specs=pl.BlockSpec((MB, D), lambda i: (i, 0)),
    out_shape=jax.ShapeDtypeStruct((N, D), jnp.float32),
)


def _mid_body(s1_ref, hn1_ref, hist_ref, b1_ref, w2_ref, o_ref):
    dinv = lax.rsqrt(jnp.sum(hist_ref[...], axis=1, keepdims=True) + 1.0)
    t = dinv * (s1_ref[0] + s1_ref[1] + hn1_ref[...]) + b1_ref[...]
    t = jnp.maximum(t, 0.0)
    o_ref[...] = dinv * jnp.dot(t, w2_ref[...], preferred_element_type=jnp.float32)


_mid = pl.pallas_call(
    _mid_body,
    grid=(GRID,),
    in_specs=[
        pl.BlockSpec((NC, MB, D), lambda i: (0, i, 0)),
        pl.BlockSpec((MB, D), lambda i: (i, 0)),
        pl.BlockSpec((MB, NW), lambda i: (i, 0)),
        pl.BlockSpec((1, D), lambda i: (0, 0)),
        pl.BlockSpec((D, D), lambda i: (0, 0)),
    ],
    out_specs=pl.BlockSpec((MB, D), lambda i: (i, 0)),
    out_shape=jax.ShapeDtypeStruct((N, D), jnp.float32),
)


def _fin_body(s2_ref, hn2_ref, hist_ref, b2_ref, o_ref):
    dinv = lax.rsqrt(jnp.sum(hist_ref[...], axis=1, keepdims=True) + 1.0)
    o_ref[...] = dinv * (s2_ref[0] + s2_ref[1] + hn2_ref[...]) + b2_ref[...]


_fin = pl.pallas_call(
    _fin_body,
    grid=(GRID,),
    in_specs=[
        pl.BlockSpec((NC, MB, D), lambda i: (0, i, 0)),
        pl.BlockSpec((MB, D), lambda i: (i, 0)),
        pl.BlockSpec((MB, NW), lambda i: (i, 0)),
        pl.BlockSpec((1, D), lambda i: (0, 0)),
    ],
    out_specs=pl.BlockSpec((MB, D), lambda i: (i, 0)),
    out_shape=jax.ShapeDtypeStruct((N, D), jnp.float32),
)


def _deg_jnp(dst_p):
    cnt = jax.ops.segment_sum(jnp.ones((E_PAD,), jnp.float32), dst_p,
                              num_segments=ACC)
    hist = jnp.zeros((NC, ACC, 16), jnp.float32)
    return hist.at[0, :, 0].set(cnt)


def _scat_jnp(hn, src_p, dst_p):
    s = jax.ops.segment_sum(hn[src_p], dst_p, num_segments=ACC)
    return jnp.stack([s, jnp.zeros_like(s)])


def kernel(x, edge_index, W1, b1, W2, b2):
    ei = edge_index.astype(jnp.int32)
    pad = E_PAD - E
    src_p = jnp.concatenate([ei[0], jnp.zeros((pad,), jnp.int32)])
    dst_p = jnp.concatenate([ei[1], jnp.full((pad,), N, jnp.int32)])
    b1r = b1.reshape(1, D)
    b2r = b2.reshape(1, D)

    hist = _deg_kernel(dst_p).T
    hn1 = _lin1(x, W1, hist)
    s1 = _scat_kernel(hn1, src_p, dst_p)
    hn2 = _mid(s1, hn1, hist, b1r, W2)
    s2 = _scat_kernel(hn2, src_p, dst_p)
    return _fin(s2, hn2, hist, b2r)
